# baseline (speedup 1.0000x reference)
import functools

import jax
import jax.numpy as jnp
from jax.experimental import pallas as pl
from jax.experimental.pallas import tpu as pltpu

_TEMP = 0.5
_SCALE = 1.0 / (_TEMP + 1e-08)

_BLOCK_M = 512


def _router_body(x_ref, w_ref, b_ref, o_ref):
    logits = jax.lax.dot_general(
        x_ref[...], w_ref[...],
        dimension_numbers=(((1,), (0,)), ((), ())),
        preferred_element_type=jnp.float32,
        precision=jax.lax.Precision.HIGHEST,
    )
    logits = (logits + b_ref[...]) * _SCALE
    o_ref[...] = jax.nn.sigmoid(logits)


@functools.partial(jax.jit, static_argnames=())
def kernel(inputs, proj, logit_bias):
    tokens, d_model = inputs.shape
    units = proj.shape[1]
    grid = (tokens // _BLOCK_M,)
    bias2d = logit_bias.reshape(1, units)
    return pl.pallas_call(
        _router_body,
        grid=grid,
        in_specs=[
            pl.BlockSpec((_BLOCK_M, d_model), lambda i: (i, 0)),
            pl.BlockSpec((d_model, units), lambda i: (0, 0)),
            pl.BlockSpec((1, units), lambda i: (0, 0)),
        ],
        out_specs=pl.BlockSpec((_BLOCK_M, units), lambda i: (i, 0)),
        out_shape=jax.ShapeDtypeStruct((tokens, units), jnp.float32),
        compiler_params=pltpu.CompilerParams(
            dimension_semantics=("arbitrary",),
        ),
    )(inputs, proj, bias2d)
